# baseline (speedup 1.0000x reference)
import functools

import jax
import jax.numpy as jnp
from jax import lax
from jax.experimental import pallas as pl
from jax.experimental.pallas import tpu as pltpu
from jax.experimental.pallas import tpu_sc as plsc

BATCH = 16384
NC = 2
NS = 16
NW = NC * NS
BPW = BATCH // NW
CHUNK = 128
NCHUNK = BPW // CHUNK
NGROUP = CHUNK // 16

_GATHER_DNUMS = lax.GatherDimensionNumbers(
    offset_dims=(), collapsed_slice_dims=(0,), start_index_map=(0,))


def _lane_gather(v, ids):
    return lax.gather(v, ids[:, None], _GATHER_DNUMS, (1,),
                      mode=lax.GatherScatterMode.PROMISE_IN_BOUNDS)


def _tp_block(x, b, d):
    w = x.shape[1]
    if d == 64:
        y = jnp.transpose(x, (1, 0)).reshape(w // 2, 2, 64)
        return jnp.concatenate([y[:, 0, :], y[:, 1, :]], axis=1)
    blk = jnp.concatenate(
        [x, jnp.zeros((64 - d, w), jnp.float32),
         jnp.ones((1, w), jnp.float32), b], axis=0)
    y = jnp.transpose(blk, (1, 0))
    return jnp.concatenate([y, jnp.zeros((w, 128 - y.shape[1]), jnp.float32)],
                           axis=1)


def _make_tp_body(d, w, nmain, extra_off, extra_w, tail_w, rows, rdiv):
    def body(x_hbm, b_hbm, xt_ref, bt_ref, o_hbm,
             xv0, xv1, bv0, bv1, yv0, yv1,
             ia, ib_, oa, ob):
        def start_in(off, width, xv, bv, sem):
            pltpu.make_async_copy(x_hbm.at[:, pl.ds(off, width)], xv, sem).start()
            pltpu.make_async_copy(b_hbm.at[:, pl.ds(off, width)], bv, sem).start()

        def wait_in(off, width, xv, bv, sem):
            pltpu.make_async_copy(x_hbm.at[:, pl.ds(off, width)], xv, sem).wait()
            pltpu.make_async_copy(b_hbm.at[:, pl.ds(off, width)], bv, sem).wait()

        def out_copy(off, width, yv, sem):
            o = off // rdiv
            if not isinstance(o, int):
                o = pl.multiple_of(o, 8)
            return pltpu.make_async_copy(
                yv, o_hbm.at[pl.ds(o, width // rdiv), :], sem)

        start_in(0, w, xv0, bv0, ia)
        nhalf = nmain // 2

        def step(k, _):
            o0 = pl.multiple_of(2 * k * w, 128)
            o1 = pl.multiple_of((2 * k + 1) * w, 128)
            o2 = pl.multiple_of((2 * k + 2) * w, 128)
            start_in(o1, w, xv1, bv1, ib_)
            wait_in(o0, w, xv0, bv0, ia)

            @pl.when(k > 0)
            def _():
                out_copy(0, w, yv0, oa).wait()

            yv0[...] = _tp_block(xv0[...], bv0[...], d)
            out_copy(o0, w, yv0, oa).start()

            @pl.when(k < nhalf - 1)
            def _():
                start_in(o2, w, xv0, bv0, ia)

            wait_in(o1, w, xv1, bv1, ib_)

            @pl.when(k > 0)
            def _():
                out_copy(0, w, yv1, ob).wait()

            yv1[...] = _tp_block(xv1[...], bv1[...], d)
            out_copy(o1, w, yv1, ob).start()
            return 0

        lax.fori_loop(0, nhalf, step, 0)
        out_copy(0, w, yv0, oa).wait()
        out_copy(0, w, yv1, ob).wait()

        if extra_w:
            start_in(extra_off, extra_w, xv0.at[:, pl.ds(0, extra_w)],
                     bv0.at[:, pl.ds(0, extra_w)], ia)
            wait_in(extra_off, extra_w, xv0.at[:, pl.ds(0, extra_w)],
                    bv0.at[:, pl.ds(0, extra_w)], ia)
            yv0[pl.ds(0, extra_w // rdiv), :] = _tp_block(
                xv0[:, pl.ds(0, extra_w)], bv0[:, pl.ds(0, extra_w)], d)
            cpx = out_copy(extra_off, extra_w,
                           yv0.at[pl.ds(0, extra_w // rdiv), :], oa)
            cpx.start()
            cpx.wait()
        yv1[pl.ds(0, tail_w // rdiv), :] = _tp_block(xt_ref[...], bt_ref[...], d)
        cpt = out_copy(rows - tail_w, tail_w,
                       yv1.at[pl.ds(0, tail_w // rdiv), :], ob)
        cpt.start()
        cpt.wait()

    return body


def _transpose_table(wT, bT, d, rows, w, nmain, extra_off, extra_w, tail_w,
                     rdiv=1):
    xt = lax.slice(wT, (0, rows - tail_w), (d, rows))
    bt = lax.slice(bT, (0, rows - tail_w), (1, rows))
    body = _make_tp_body(d, w, nmain, extra_off, extra_w, tail_w, rows, rdiv)
    return pl.pallas_call(
        body,
        in_specs=[pl.BlockSpec(memory_space=pl.ANY),
                  pl.BlockSpec(memory_space=pl.ANY),
                  pl.BlockSpec(memory_space=pltpu.VMEM),
                  pl.BlockSpec(memory_space=pltpu.VMEM)],
        out_specs=pl.BlockSpec(memory_space=pl.ANY),
        out_shape=jax.ShapeDtypeStruct((rows // rdiv, 128), jnp.float32),
        scratch_shapes=[pltpu.VMEM((d, w), jnp.float32),
                        pltpu.VMEM((d, w), jnp.float32),
                        pltpu.VMEM((1, w), jnp.float32),
                        pltpu.VMEM((1, w), jnp.float32),
                        pltpu.VMEM((w // rdiv, 128), jnp.float32),
                        pltpu.VMEM((w // rdiv, 128), jnp.float32),
                        pltpu.SemaphoreType.DMA,
                        pltpu.SemaphoreType.DMA,
                        pltpu.SemaphoreType.DMA,
                        pltpu.SemaphoreType.DMA],
    )(wT, bT, xt, bt)


def _mf_body(user_hbm, item_hbm, cat_hbm, brand_hbm,
             user_w, item_w, cat_w, brand_w,
             ubias_w, ibias_w, gbias_hbm,
             y_hbm, l2_hbm,
             uidx_v, iidx_v, cidx_v, bidx_v, uq_v,
             urows_v, irows_v, crows_v, brows_v,
             ub_v, ib_v, gb_v, y_v, l2_v, sem):
    wid = lax.axis_index("s") * NC + lax.axis_index("c")

    pltpu.sync_copy(user_hbm.at[wid], uidx_v)
    pltpu.sync_copy(item_hbm.at[wid], iidx_v)
    pltpu.sync_copy(cat_hbm.at[wid], cidx_v)
    pltpu.sync_copy(brand_hbm.at[wid], bidx_v)
    pltpu.sync_copy(gbias_hbm, gb_v)

    for c in range(NCHUNK):
        for k in range(CHUNK // 16):
            sl = pl.ds(k * 16, 16)
            uq_v[c, sl] = jnp.right_shift(uidx_v[c, sl], 1)

    bias_copies = []
    for c in range(NCHUNK):
        sl = pl.ds(c * CHUNK, CHUNK)
        bias_copies.append(pltpu.async_copy(ubias_w.at[uidx_v.at[c]], ub_v.at[sl], sem))
        bias_copies.append(pltpu.async_copy(ibias_w.at[iidx_v.at[c]], ib_v.at[sl], sem))
    for cp in bias_copies:
        cp.wait()

    gb = gb_v[...]
    lane = lax.iota(jnp.int32, 16)
    rot_ids = [(lane + sh) % 16 for sh in (8, 4, 2, 1)]
    lane_masks = [lane == j for j in range(16)]

    def chunk_body(c, l2acc_c):
        copies = [pltpu.async_copy(user_w.at[uq_v.at[c]], urows_v, sem),
                  pltpu.async_copy(item_w.at[iidx_v.at[c]], irows_v, sem),
                  pltpu.async_copy(cat_w.at[cidx_v.at[c]], crows_v, sem),
                  pltpu.async_copy(brand_w.at[bidx_v.at[c]], brows_v, sem)]
        for cp in copies:
            cp.wait()

        def group_body(g, l2acc):
            base = g * 16
            puv = jnp.bitwise_and(uidx_v[c, pl.ds(base, 16)], 1)
            dots = jnp.zeros((16,), jnp.float32)
            for j in range(16):
                e = base + j
                pu = puv[j]
                uh = [urows_v[e, pl.ds(h * 16, 16)] for h in range(8)]
                u0 = jnp.where(pu == 1, uh[4], uh[0])
                u1 = jnp.where(pu == 1, uh[5], uh[1])
                u2 = jnp.where(pu == 1, uh[6], uh[2])
                u3 = jnp.where(pu == 1, uh[7], uh[3])
                v0 = irows_v[e, pl.ds(0, 16)]
                v1 = irows_v[e, pl.ds(16, 16)]
                v2 = crows_v[e, pl.ds(0, 16)]
                v3 = brows_v[e, pl.ds(0, 16)]
                p = u0 * v0 + u1 * v1 + u2 * v2 + u3 * v3
                l2acc = (l2acc + u0 * u0 + u1 * u1 + u2 * u2 + u3 * u3
                         + v0 * v0 + v1 * v1 + v2 * v2 + v3 * v3)
                for ids in rot_ids:
                    p = p + _lane_gather(p, ids)
                dots = jnp.where(lane_masks[j], p, dots)
            off = c * CHUNK + base
            y_v[pl.ds(off, 16)] = (dots + ub_v[pl.ds(off, 16)]
                                   + ib_v[pl.ds(off, 16)] + gb)
            return l2acc

        return lax.fori_loop(0, NGROUP, group_body, l2acc_c)

    l2_v[...] = lax.fori_loop(0, NCHUNK, chunk_body,
                              jnp.zeros((16,), jnp.float32))

    pltpu.sync_copy(y_v, y_hbm.at[wid])
    pltpu.sync_copy(l2_v, l2_hbm.at[wid])


@jax.jit
def _mf_kernel(user, item, item_cat, item_brand,
               user_emb_w, item_emb_w, cat_emb_w, brand_emb_w,
               user_bias_w, item_bias_w, gbias16):
    user_w = _transpose_table(user_emb_w.T, user_bias_w.T, 64, 1000000,
                              15616, 64, 999424, 512, 64, rdiv=2)
    item_w = _transpose_table(item_emb_w.T, item_bias_w.T, 32, 100000,
                              16384, 6, 98304, 1664, 32)
    cat_w = jnp.pad(cat_emb_w, ((0, 0), (0, 112)))
    brand_w = jnp.pad(brand_emb_w, ((0, 0), (0, 112)))

    mesh = plsc.VectorSubcoreMesh(core_axis_name="c", subcore_axis_name="s")
    run = functools.partial(
        pl.kernel,
        mesh=mesh,
        compiler_params=pltpu.CompilerParams(use_tc_tiling_on_sc=True),
        out_type=[
            jax.ShapeDtypeStruct((NW, BPW), jnp.float32),
            jax.ShapeDtypeStruct((NW, 16), jnp.float32),
        ],
        scratch_types=[
            pltpu.VMEM((NCHUNK, CHUNK), jnp.int32),
            pltpu.VMEM((NCHUNK, CHUNK), jnp.int32),
            pltpu.VMEM((NCHUNK, CHUNK), jnp.int32),
            pltpu.VMEM((NCHUNK, CHUNK), jnp.int32),
            pltpu.VMEM((NCHUNK, CHUNK), jnp.int32),
            pltpu.VMEM((CHUNK, 128), jnp.float32),
            pltpu.VMEM((CHUNK, 128), jnp.float32),
            pltpu.VMEM((CHUNK, 128), jnp.float32),
            pltpu.VMEM((CHUNK, 128), jnp.float32),
            pltpu.VMEM((BPW,), jnp.float32),
            pltpu.VMEM((BPW,), jnp.float32),
            pltpu.VMEM((16,), jnp.float32),
            pltpu.VMEM((BPW,), jnp.float32),
            pltpu.VMEM((16,), jnp.float32),
            pltpu.SemaphoreType.DMA,
        ],
    )(_mf_body)
    return run(user.reshape(NW, NCHUNK, CHUNK),
               item.reshape(NW, NCHUNK, CHUNK),
               item_cat.reshape(NW, NCHUNK, CHUNK),
               item_brand.reshape(NW, NCHUNK, CHUNK),
               user_w, item_w, cat_w, brand_w,
               user_bias_w.reshape(-1), item_bias_w.reshape(-1), gbias16)


def kernel(user, item, item_cat, item_brand, user_emb_w, item_emb_w,
           cat_emb_w, brand_emb_w, user_bias_w, item_bias_w, global_bias):
    gbias16 = jnp.broadcast_to(global_bias.astype(jnp.float32), (16,))
    y, l2p = _mf_kernel(user.astype(jnp.int32), item.astype(jnp.int32),
                        item_cat.astype(jnp.int32), item_brand.astype(jnp.int32),
                        user_emb_w, item_emb_w, cat_emb_w, brand_emb_w,
                        user_bias_w, item_bias_w, gbias16)
    return y.reshape(BATCH), jnp.sum(l2p) / BATCH / 2


# speedup vs baseline: 1.6385x; 1.6385x over previous
import functools

import jax
import jax.numpy as jnp
from jax import lax
from jax.experimental import pallas as pl
from jax.experimental.pallas import tpu as pltpu
from jax.experimental.pallas import tpu_sc as plsc

BATCH = 16384
NC = 2
NS = 16
NW = NC * NS
BPW = BATCH // NW
CHUNK = 128
NCHUNK = BPW // CHUNK
NGROUP = CHUNK // 16

_GATHER_DNUMS = lax.GatherDimensionNumbers(
    offset_dims=(), collapsed_slice_dims=(0,), start_index_map=(0,))


def _lane_gather(v, ids):
    return lax.gather(v, ids[:, None], _GATHER_DNUMS, (1,),
                      mode=lax.GatherScatterMode.PROMISE_IN_BOUNDS)


def _tp_block(x, b, d):
    w = x.shape[1]
    if d == 64:
        blk = jnp.concatenate([x, b, jnp.ones((1, w), jnp.float32)], axis=0)
    else:
        blk = jnp.concatenate(
            [x, jnp.zeros((64 - d, w), jnp.float32),
             jnp.ones((1, w), jnp.float32), b], axis=0)
    y = jnp.transpose(blk, (1, 0))
    return jnp.concatenate([y, jnp.zeros((w, 128 - y.shape[1]), jnp.float32)],
                           axis=1)


def _make_tp_body(d, w, nmain, extra_off, extra_w, tail_w, rows):
    def body(x_hbm, b_hbm, xt_ref, bt_ref, o_hbm,
             xv0, xv1, bv0, bv1, yv0, yv1,
             ia, ib_, oa, ob):
        def start_in(off, width, xv, bv, sem):
            pltpu.make_async_copy(x_hbm.at[:, pl.ds(off, width)], xv, sem).start()
            pltpu.make_async_copy(b_hbm.at[:, pl.ds(off, width)], bv, sem).start()

        def wait_in(off, width, xv, bv, sem):
            pltpu.make_async_copy(x_hbm.at[:, pl.ds(off, width)], xv, sem).wait()
            pltpu.make_async_copy(b_hbm.at[:, pl.ds(off, width)], bv, sem).wait()

        def out_copy(off, width, yv, sem):
            return pltpu.make_async_copy(
                yv, o_hbm.at[pl.ds(off, width), :], sem)

        start_in(0, w, xv0, bv0, ia)
        nhalf = nmain // 2

        def step(k, _):
            o0 = pl.multiple_of(2 * k * w, 128)
            o1 = pl.multiple_of((2 * k + 1) * w, 128)
            o2 = pl.multiple_of((2 * k + 2) * w, 128)
            start_in(o1, w, xv1, bv1, ib_)
            wait_in(o0, w, xv0, bv0, ia)

            @pl.when(k > 0)
            def _():
                out_copy(0, w, yv0, oa).wait()

            yv0[...] = _tp_block(xv0[...], bv0[...], d)
            out_copy(o0, w, yv0, oa).start()

            @pl.when(k < nhalf - 1)
            def _():
                start_in(o2, w, xv0, bv0, ia)

            wait_in(o1, w, xv1, bv1, ib_)

            @pl.when(k > 0)
            def _():
                out_copy(0, w, yv1, ob).wait()

            yv1[...] = _tp_block(xv1[...], bv1[...], d)
            out_copy(o1, w, yv1, ob).start()
            return 0

        lax.fori_loop(0, nhalf, step, 0)
        out_copy(0, w, yv0, oa).wait()
        out_copy(0, w, yv1, ob).wait()

        if extra_w:
            start_in(extra_off, extra_w, xv0.at[:, pl.ds(0, extra_w)],
                     bv0.at[:, pl.ds(0, extra_w)], ia)
            wait_in(extra_off, extra_w, xv0.at[:, pl.ds(0, extra_w)],
                    bv0.at[:, pl.ds(0, extra_w)], ia)
            yv0[pl.ds(0, extra_w), :] = _tp_block(
                xv0[:, pl.ds(0, extra_w)], bv0[:, pl.ds(0, extra_w)], d)
            cpx = out_copy(extra_off, extra_w, yv0.at[pl.ds(0, extra_w), :], oa)
            cpx.start()
            cpx.wait()
        yv1[pl.ds(0, tail_w), :] = _tp_block(xt_ref[...], bt_ref[...], d)
        cpt = out_copy(rows - tail_w, tail_w, yv1.at[pl.ds(0, tail_w), :], ob)
        cpt.start()
        cpt.wait()

    return body


def _transpose_table(wT, bT, d, rows, w, nmain, extra_off, extra_w, tail_w):
    xt = lax.slice(wT, (0, rows - tail_w), (d, rows))
    bt = lax.slice(bT, (0, rows - tail_w), (1, rows))
    body = _make_tp_body(d, w, nmain, extra_off, extra_w, tail_w, rows)
    return pl.pallas_call(
        body,
        in_specs=[pl.BlockSpec(memory_space=pl.ANY),
                  pl.BlockSpec(memory_space=pl.ANY),
                  pl.BlockSpec(memory_space=pltpu.VMEM),
                  pl.BlockSpec(memory_space=pltpu.VMEM)],
        out_specs=pl.BlockSpec(memory_space=pl.ANY),
        out_shape=jax.ShapeDtypeStruct((rows, 128), jnp.float32),
        scratch_shapes=[pltpu.VMEM((d, w), jnp.float32),
                        pltpu.VMEM((d, w), jnp.float32),
                        pltpu.VMEM((1, w), jnp.float32),
                        pltpu.VMEM((1, w), jnp.float32),
                        pltpu.VMEM((w, 128), jnp.float32),
                        pltpu.VMEM((w, 128), jnp.float32),
                        pltpu.SemaphoreType.DMA,
                        pltpu.SemaphoreType.DMA,
                        pltpu.SemaphoreType.DMA,
                        pltpu.SemaphoreType.DMA],
    )(wT, bT, xt, bt)


def _mf_body(user_hbm, item_hbm, cat_hbm, brand_hbm,
             user_w, item_w, cat_w, brand_w, gbias_hbm,
             y_hbm, l2_hbm,
             uidx_v, iidx_v, cidx_v, bidx_v,
             urows_v, irows_v, crows_v, brows_v,
             gb_v, y_v, l2_v, sem):
    wid = lax.axis_index("s") * NC + lax.axis_index("c")

    pltpu.sync_copy(user_hbm.at[wid], uidx_v)
    pltpu.sync_copy(item_hbm.at[wid], iidx_v)
    pltpu.sync_copy(cat_hbm.at[wid], cidx_v)
    pltpu.sync_copy(brand_hbm.at[wid], bidx_v)
    pltpu.sync_copy(gbias_hbm, gb_v)

    gb = gb_v[...]
    lane = lax.iota(jnp.int32, 16)
    rot_ids = [(lane + sh) % 16 for sh in (8, 4, 2, 1)]
    lane_masks = [lane == j for j in range(16)]

    def chunk_body(c, l2acc_c):
        copies = [pltpu.async_copy(user_w.at[uidx_v.at[c]], urows_v, sem),
                  pltpu.async_copy(item_w.at[iidx_v.at[c]], irows_v, sem),
                  pltpu.async_copy(cat_w.at[cidx_v.at[c]], crows_v, sem),
                  pltpu.async_copy(brand_w.at[bidx_v.at[c]], brows_v, sem)]
        for cp in copies:
            cp.wait()

        def group_body(g, l2acc):
            base = g * 16
            dots = jnp.zeros((16,), jnp.float32)
            for j in range(16):
                e = base + j
                u0 = urows_v[e, pl.ds(0, 16)]
                u1 = urows_v[e, pl.ds(16, 16)]
                u2 = urows_v[e, pl.ds(32, 16)]
                u3 = urows_v[e, pl.ds(48, 16)]
                u4 = urows_v[e, pl.ds(64, 16)]
                v0 = irows_v[e, pl.ds(0, 16)]
                v1 = irows_v[e, pl.ds(16, 16)]
                v4 = irows_v[e, pl.ds(64, 16)]
                v2 = crows_v[e, pl.ds(0, 16)]
                v3 = brows_v[e, pl.ds(0, 16)]
                p = u0 * v0 + u1 * v1 + u2 * v2 + u3 * v3 + u4 * v4
                l2acc = (l2acc + u0 * u0 + u1 * u1 + u2 * u2 + u3 * u3
                         + v0 * v0 + v1 * v1 + v2 * v2 + v3 * v3)
                for ids in rot_ids:
                    p = p + _lane_gather(p, ids)
                dots = jnp.where(lane_masks[j], p, dots)
            y_v[pl.ds(c * CHUNK + base, 16)] = dots + gb
            return l2acc

        return lax.fori_loop(0, NGROUP, group_body, l2acc_c)

    l2_v[...] = lax.fori_loop(0, NCHUNK, chunk_body,
                              jnp.zeros((16,), jnp.float32))

    pltpu.sync_copy(y_v, y_hbm.at[wid])
    pltpu.sync_copy(l2_v, l2_hbm.at[wid])


@jax.jit
def _mf_kernel(user, item, item_cat, item_brand,
               user_emb_w, item_emb_w, cat_emb_w, brand_emb_w,
               user_bias_w, item_bias_w, gbias16):
    user_w = _transpose_table(user_emb_w.T, user_bias_w.T, 64, 1000000,
                              15616, 64, 999424, 512, 64)
    item_w = _transpose_table(item_emb_w.T, item_bias_w.T, 32, 100000,
                              16384, 6, 98304, 1664, 32)
    cat_w = jnp.pad(cat_emb_w, ((0, 0), (0, 112)))
    brand_w = jnp.pad(brand_emb_w, ((0, 0), (0, 112)))

    mesh = plsc.VectorSubcoreMesh(core_axis_name="c", subcore_axis_name="s")
    run = functools.partial(
        pl.kernel,
        mesh=mesh,
        compiler_params=pltpu.CompilerParams(use_tc_tiling_on_sc=True),
        out_type=[
            jax.ShapeDtypeStruct((NW, BPW), jnp.float32),
            jax.ShapeDtypeStruct((NW, 16), jnp.float32),
        ],
        scratch_types=[
            pltpu.VMEM((NCHUNK, CHUNK), jnp.int32),
            pltpu.VMEM((NCHUNK, CHUNK), jnp.int32),
            pltpu.VMEM((NCHUNK, CHUNK), jnp.int32),
            pltpu.VMEM((NCHUNK, CHUNK), jnp.int32),
            pltpu.VMEM((CHUNK, 128), jnp.float32),
            pltpu.VMEM((CHUNK, 128), jnp.float32),
            pltpu.VMEM((CHUNK, 128), jnp.float32),
            pltpu.VMEM((CHUNK, 128), jnp.float32),
            pltpu.VMEM((16,), jnp.float32),
            pltpu.VMEM((BPW,), jnp.float32),
            pltpu.VMEM((16,), jnp.float32),
            pltpu.SemaphoreType.DMA,
        ],
    )(_mf_body)
    return run(user.reshape(NW, NCHUNK, CHUNK),
               item.reshape(NW, NCHUNK, CHUNK),
               item_cat.reshape(NW, NCHUNK, CHUNK),
               item_brand.reshape(NW, NCHUNK, CHUNK),
               user_w, item_w, cat_w, brand_w, gbias16)


def kernel(user, item, item_cat, item_brand, user_emb_w, item_emb_w,
           cat_emb_w, brand_emb_w, user_bias_w, item_bias_w, global_bias):
    gbias16 = jnp.broadcast_to(global_bias.astype(jnp.float32), (16,))
    y, l2p = _mf_kernel(user.astype(jnp.int32), item.astype(jnp.int32),
                        item_cat.astype(jnp.int32), item_brand.astype(jnp.int32),
                        user_emb_w, item_emb_w, cat_emb_w, brand_emb_w,
                        user_bias_w, item_bias_w, gbias16)
    return y.reshape(BATCH), jnp.sum(l2p) / BATCH / 2
